# baseline (speedup 1.0000x reference)
import jax
import jax.numpy as jnp
from jax import lax
from jax.experimental import pallas as pl
from jax.experimental.pallas import tpu as pltpu
from jax.experimental.pallas import tpu_sc as plsc

VOCAB = 1000000
EMBED_DIM = 32
BATCH = 4096
HIST = 200

NC = 2
NS = 16
NW = NC * NS
BPW = BATCH // NW
DT = EMBED_DIM // 8
CH = 4
NBUF = 2
L = 16
TCH = 800
TGC = VOCAB // TCH
TK = TGC // NW
TREM = TGC - TK * NW


def _tp_body(tt_hbm, scr_hbm, in_v, out_v, gsems, wsems):
    wid = lax.axis_index("s") * NC + lax.axis_index("c")
    lanes = lax.iota(jnp.int32, L)

    def chunk_r0(c):
        return (wid + c * NW) * TCH

    def fire_read(c, b):
        pltpu.async_copy(
            tt_hbm.at[:, pl.ds(chunk_r0(c), TCH)], in_v.at[b], gsems.at[b]
        )

    def drain_read(b):
        pltpu.make_async_copy(
            tt_hbm.at[:, pl.ds(0, TCH)], in_v.at[b], gsems.at[b]
        ).wait()

    def transpose(b):
        @plsc.parallel_loop(0, TCH, step=1, unroll=8)
        def _r(r):
            r_vec = jnp.full((L,), 0, jnp.int32) + r
            for k in range(EMBED_DIM // L):
                vals = plsc.load_gather(in_v.at[b], [lanes + k * L, r_vec])
                out_v[b, r, pl.ds(k * L, L)] = vals

    def fire_write(c, b):
        pltpu.async_copy(
            out_v.at[b], scr_hbm.at[pl.ds(chunk_r0(c), TCH)], wsems.at[b]
        )

    def drain_write(b):
        pltpu.make_async_copy(
            out_v.at[b], scr_hbm.at[pl.ds(0, TCH)], wsems.at[b]
        ).wait()

    nch = TK + jnp.where(wid < TREM, 1, 0)

    for b in range(NBUF):
        fire_read(b, b)

    @pl.loop(0, TK + 1, step=NBUF)
    def _chunks(g):
        for b in range(NBUF):
            c = g + b

            @pl.when(c < nch)
            def _():
                drain_read(b)

                @pl.when(c >= NBUF)
                def _():
                    drain_write(b)

                transpose(b)
                fire_write(c, b)

                @pl.when(c + NBUF < nch)
                def _():
                    fire_read(c + NBUF, b)

    for b in range(NBUF):
        drain_write(b)


def _emb_body(xt_hbm, table_hbm, out_hbm, idxt_v, idx1d_v, rows_v, tr_v, gsems, wsems):
    wid = lax.axis_index("s") * NC + lax.axis_index("c")
    b0 = wid * BPW

    pltpu.sync_copy(xt_hbm.at[:, pl.ds(b0, BPW)], idxt_v)

    lanes = lax.iota(jnp.int32, L)
    l_vecs = [lanes + (k * L) for k in range(BPW // L)]

    @plsc.parallel_loop(0, HIST, step=1, unroll=8)
    def _flatten(h):
        for k in range(BPW // L):
            idx1d_v[pl.ds(h * BPW + k * L, L)] = idxt_v[h, pl.ds(k * L, L)]

    def fire_gather(c, b):
        pltpu.async_copy(
            table_hbm.at[idx1d_v.at[pl.ds(c * CH * BPW, CH * BPW)]],
            rows_v.at[b],
            gsems.at[b],
        )

    def drain_gather(b):
        pltpu.make_async_copy(
            table_hbm.at[pl.ds(0, CH * BPW)], rows_v.at[b], gsems.at[b]
        ).wait()

    def transpose(b):
        @plsc.parallel_loop(0, CH * EMBED_DIM, step=1, unroll=8)
        def _i(i):
            hh = i // EMBED_DIM
            d = lax.rem(i, EMBED_DIM)
            d_vec = jnp.full((L,), 0, jnp.int32) + d
            dt = d // 8
            ds = lax.rem(d, 8)
            row0 = hh * BPW
            for k in range(BPW // L):
                vals = plsc.load_gather(rows_v.at[b], [l_vecs[k] + row0, d_vec])
                tr_v[b, hh, dt, ds, pl.ds(k * L, L)] = vals

    def fire_write(c, b):
        pltpu.async_copy(
            tr_v.at[b], out_hbm.at[pl.ds(c * CH, CH)].at[:, :, wid], wsems.at[b]
        )

    def drain_write(b):
        pltpu.make_async_copy(
            tr_v.at[b], out_hbm.at[pl.ds(0, CH)].at[:, :, 0], wsems.at[b]
        ).wait()

    for b in range(NBUF):
        fire_gather(b, b)

    @pl.loop(0, HIST // CH, step=NBUF)
    def _steps(g):
        for b in range(NBUF):
            c = g + b
            drain_gather(b)

            @pl.when(c >= NBUF)
            def _():
                drain_write(b)

            transpose(b)
            fire_write(c, b)

            @pl.when(c + NBUF < HIST // CH)
            def _():
                fire_gather(c + NBUF, b)

    for b in range(NBUF):
        drain_write(b)


@jax.jit
def _emb_call(xt, tt):
    mesh = plsc.VectorSubcoreMesh(core_axis_name="c", subcore_axis_name="s")
    params = pltpu.CompilerParams(
        use_tc_tiling_on_sc=False, needs_layout_passes=False
    )
    scr = pl.kernel(
        _tp_body,
        out_type=jax.ShapeDtypeStruct((VOCAB, EMBED_DIM), jnp.float32),
        mesh=mesh,
        scratch_types=[
            pltpu.VMEM((NBUF, EMBED_DIM, TCH), jnp.float32),
            pltpu.VMEM((NBUF, TCH, EMBED_DIM), jnp.float32),
            pltpu.SemaphoreType.DMA((NBUF,)),
            pltpu.SemaphoreType.DMA((NBUF,)),
        ],
        compiler_params=params,
    )(tt)
    out = pl.kernel(
        _emb_body,
        out_type=jax.ShapeDtypeStruct((HIST, DT, NW, 8, BPW), jnp.float32),
        mesh=mesh,
        scratch_types=[
            pltpu.VMEM((HIST, BPW), jnp.int32),
            pltpu.VMEM((HIST * BPW,), jnp.int32),
            pltpu.VMEM((NBUF, CH * BPW, EMBED_DIM), jnp.float32),
            pltpu.VMEM((NBUF, CH, DT, 8, BPW), jnp.float32),
            pltpu.SemaphoreType.DMA((NBUF,)),
            pltpu.SemaphoreType.DMA((NBUF,)),
        ],
        compiler_params=params,
    )(xt, scr)
    return out


def kernel(x, table):
    k = _emb_call(x.astype(jnp.int32).T, table.T)
    return k.transpose(2, 4, 0, 1, 3).reshape(BATCH, HIST, EMBED_DIM)


# speedup vs baseline: 3.9215x; 3.9215x over previous
import jax
import jax.numpy as jnp
from jax import lax
from jax.experimental import pallas as pl
from jax.experimental.pallas import tpu as pltpu
from jax.experimental.pallas import tpu_sc as plsc

VOCAB = 1000000
EMBED_DIM = 32
BATCH = 4096
HIST = 200

NC = 2
NS = 16
NW = NC * NS
BPW = BATCH // NW
DT = EMBED_DIM // 8
CH = 4
NBUF = 2
L = 16


def _emb_body(xt_hbm, table_hbm, out_hbm, idxt_v, idx1d_v, rows_v, tr_v, gsems, wsems):
    wid = lax.axis_index("s") * NC + lax.axis_index("c")
    b0 = wid * BPW

    pltpu.sync_copy(xt_hbm.at[:, pl.ds(b0, BPW)], idxt_v)

    lanes = lax.iota(jnp.int32, L)
    l_vecs = [lanes + (k * L) for k in range(BPW // L)]

    @plsc.parallel_loop(0, HIST, step=1, unroll=8)
    def _flatten(h):
        for k in range(BPW // L):
            idx1d_v[pl.ds(h * BPW + k * L, L)] = idxt_v[h, pl.ds(k * L, L)]

    def fire_gather(c, b):
        pltpu.async_copy(
            table_hbm.at[idx1d_v.at[pl.ds(c * CH * BPW, CH * BPW)]],
            rows_v.at[b],
            gsems.at[b],
        )

    def drain_gather(b):
        pltpu.make_async_copy(
            table_hbm.at[pl.ds(0, CH * BPW)], rows_v.at[b], gsems.at[b]
        ).wait()

    def transpose(b):
        @plsc.parallel_loop(0, CH * EMBED_DIM, step=1, unroll=8)
        def _i(i):
            hh = i // EMBED_DIM
            d = lax.rem(i, EMBED_DIM)
            d_vec = jnp.full((L,), 0, jnp.int32) + d
            dt = d // 8
            ds = lax.rem(d, 8)
            row0 = hh * BPW
            for k in range(BPW // L):
                vals = plsc.load_gather(rows_v.at[b], [l_vecs[k] + row0, d_vec])
                tr_v[b, hh, dt, ds, pl.ds(k * L, L)] = vals

    def fire_write(c, b):
        pltpu.async_copy(
            tr_v.at[b], out_hbm.at[pl.ds(c * CH, CH)].at[:, :, wid], wsems.at[b]
        )

    def drain_write(b):
        pltpu.make_async_copy(
            tr_v.at[b], out_hbm.at[pl.ds(0, CH)].at[:, :, 0], wsems.at[b]
        ).wait()

    for b in range(NBUF):
        fire_gather(b, b)

    @pl.loop(0, HIST // CH, step=NBUF)
    def _steps(g):
        for b in range(NBUF):
            c = g + b
            drain_gather(b)

            @pl.when(c >= NBUF)
            def _():
                drain_write(b)

            transpose(b)
            fire_write(c, b)

            @pl.when(c + NBUF < HIST // CH)
            def _():
                fire_gather(c + NBUF, b)

    for b in range(NBUF):
        drain_write(b)


@jax.jit
def _emb_call(xt, table):
    mesh = plsc.VectorSubcoreMesh(core_axis_name="c", subcore_axis_name="s")
    f = pl.kernel(
        _emb_body,
        out_type=jax.ShapeDtypeStruct((HIST, DT, NW, 8, BPW), jnp.float32),
        mesh=mesh,
        scratch_types=[
            pltpu.VMEM((HIST, BPW), jnp.int32),
            pltpu.VMEM((HIST * BPW,), jnp.int32),
            pltpu.VMEM((NBUF, CH * BPW, EMBED_DIM), jnp.float32),
            pltpu.VMEM((NBUF, CH, DT, 8, BPW), jnp.float32),
            pltpu.SemaphoreType.DMA((NBUF,)),
            pltpu.SemaphoreType.DMA((NBUF,)),
        ],
        compiler_params=pltpu.CompilerParams(
            use_tc_tiling_on_sc=False, needs_layout_passes=False
        ),
    )
    return f(xt, table)


def kernel(x, table):
    k = _emb_call(x.astype(jnp.int32).T, table)
    return k.transpose(2, 4, 0, 1, 3).reshape(BATCH, HIST, EMBED_DIM)
